# baseline (speedup 1.0000x reference)
import functools

import jax
import jax.numpy as jnp
from jax import lax
from jax.experimental import pallas as pl
from jax.experimental.pallas import tpu as pltpu
from jax.experimental.pallas import tpu_sc as plsc


def _argmin_kernel(x_ref, w_ref, x2_ref, w2_ref, idx_ref, loss_ref, *, k_total):
    x = x_ref[:]
    w = w_ref[:]

    x2 = x2_ref[:]
    w2 = w2_ref[:]
    n2dot = jax.lax.dot_general(
        -2.0 * x, w, (((1,), (1,)), ((), ())),
        preferred_element_type=jnp.float32)
    d2 = x2 + w2 + n2dot
    c = jnp.maximum(d2, 0.0)
    dist = jnp.sqrt(c)

    m = jnp.min(dist, axis=1, keepdims=True)
    iota = jax.lax.broadcasted_iota(jnp.int32, dist.shape, 1)
    idx_ref[:] = jnp.min(jnp.where(dist == m, iota, k_total),
                         axis=1, keepdims=True)

    m2 = jnp.min(c, axis=1)
    loss_ref[...] = jnp.sum(m2).reshape(1, 1, 1)


def _st_kernel(x_ref, q_ref, out_ref):
    x = x_ref[:]
    q = q_ref[:, : x.shape[1]]
    out_ref[:] = x + (q - x)


def _sc_gather(table, idx, n, d):
    info = plsc.get_sparse_core_info()
    nc, ns = info.num_cores, info.num_subcores
    nw = nc * ns
    b_per_w = n // nw
    mesh = plsc.VectorSubcoreMesh(core_axis_name="c", subcore_axis_name="s")

    n_chunks = 2
    b_chunk = b_per_w // n_chunks

    @functools.partial(
        pl.kernel, mesh=mesh,
        out_type=jax.ShapeDtypeStruct((n, d), jnp.float32),
        scratch_types=[
            pltpu.VMEM((b_chunk,), jnp.int32),
            pltpu.VMEM((b_chunk, d), jnp.float32),
            pltpu.SemaphoreType.DMA,
        ],
    )
    def gather_k(table_hbm, idx_hbm, out_hbm, idx_v, rows_v, sem):
        wid = lax.axis_index("s") * nc + lax.axis_index("c")
        for ch in range(n_chunks):
            base = wid * b_per_w + ch * b_chunk
            pltpu.sync_copy(idx_hbm.at[pl.ds(base, b_chunk)], idx_v)
            pltpu.async_copy(table_hbm.at[idx_v], rows_v, sem).wait()
            pltpu.sync_copy(rows_v, out_hbm.at[pl.ds(base, b_chunk)])

    return gather_k(table, idx)


def _argmin_call(x_half, W, x2_half, w2, k, bn):
    nh, d = x_half.shape
    return pl.pallas_call(
        functools.partial(_argmin_kernel, k_total=k),
        grid=(nh // bn,),
        in_specs=[
            pl.BlockSpec((bn, d), lambda i: (i, 0)),
            pl.BlockSpec((k, d), lambda i: (0, 0)),
            pl.BlockSpec((bn, 1), lambda i: (i, 0)),
            pl.BlockSpec((1, k), lambda i: (0, 0)),
        ],
        out_specs=[
            pl.BlockSpec((bn, 1), lambda i: (i, 0)),
            pl.BlockSpec((1, 1, 1), lambda i: (i, 0, 0)),
        ],
        out_shape=[
            jax.ShapeDtypeStruct((nh, 1), jnp.int32),
            jax.ShapeDtypeStruct((nh // bn, 1, 1), jnp.float32),
        ],
        compiler_params=pltpu.CompilerParams(
            dimension_semantics=("parallel",)),
    )(x_half, W, x2_half, w2)


def _st_call(x_half, q_half, bs):
    nh, d = x_half.shape
    return pl.pallas_call(
        _st_kernel,
        grid=(nh // bs,),
        in_specs=[
            pl.BlockSpec((bs, d), lambda i: (i, 0)),
            pl.BlockSpec((bs, 128), lambda i: (i, 0)),
        ],
        out_specs=pl.BlockSpec((bs, d), lambda i: (i, 0)),
        out_shape=jax.ShapeDtypeStruct((nh, d), jnp.float32),
        compiler_params=pltpu.CompilerParams(
            dimension_semantics=("parallel",)),
    )(x_half, q_half)


def kernel(weights_flat, W):
    n, d = weights_flat.shape
    k, _ = W.shape
    bn = 1024
    nh = n // 2

    x2 = jnp.sum(weights_flat * weights_flat, axis=1, keepdims=True)
    w2 = jnp.sum(W * W, axis=1)[None, :]

    w_pad = jnp.concatenate(
        [W, jnp.zeros((k, 128 - d), jnp.float32)], axis=1)

    x_a, x_b = weights_flat[:nh], weights_flat[nh:]
    idx1, loss1 = _argmin_call(x_a, W, x2[:nh], w2, k, bn)
    q1 = _sc_gather(w_pad, idx1.reshape(nh), nh, 128)
    idx2, loss2 = _argmin_call(x_b, W, x2[nh:], w2, k, bn)
    q2 = _sc_gather(w_pad, idx2.reshape(nh), nh, 128)
    out1 = _st_call(x_a, q1, 2048)
    out2 = _st_call(x_b, q2, 2048)
    out = jnp.concatenate([out1, out2], axis=0)

    mean_sq = (jnp.sum(loss1) + jnp.sum(loss2)) / (n * d)
    vq_loss = mean_sq + 0.1 * mean_sq
    return (out, vq_loss)


# speedup vs baseline: 1.3283x; 1.3283x over previous
import functools

import jax
import jax.numpy as jnp
from jax.experimental import pallas as pl
from jax.experimental.pallas import tpu as pltpu


def _vq_kernel(x_ref, w_ref, x2_ref, w2_ref, out_ref, loss_ref, *, k_total):
    x = x_ref[:]
    w = w_ref[:]

    x2 = x2_ref[:]
    w2 = w2_ref[:]
    n2dot = jax.lax.dot_general(
        -2.0 * x, w, (((1,), (1,)), ((), ())),
        preferred_element_type=jnp.float32)
    d2 = x2 + w2 + n2dot
    dist = jnp.sqrt(jnp.maximum(d2, 0.0))

    m = jnp.min(dist, axis=1, keepdims=True)
    iota = jax.lax.broadcasted_iota(jnp.int32, dist.shape, 1)
    idx = jnp.min(jnp.where(dist == m, iota, k_total), axis=1, keepdims=True)
    onehot = (iota == idx).astype(jnp.float32)

    q = jax.lax.dot_general(
        onehot, w, (((1,), (0,)), ((), ())),
        preferred_element_type=jnp.float32)

    out_ref[:] = x + (q - x)

    loss_ref[...] = jnp.sum(m * m).reshape(1, 1, 1)


def kernel(weights_flat, W):
    n, d = weights_flat.shape
    k, _ = W.shape
    bn = 1024
    grid = (n // bn,)

    x2 = jnp.sum(weights_flat * weights_flat, axis=1, keepdims=True)
    w2 = jnp.sum(W * W, axis=1)[None, :]

    out, loss_parts = pl.pallas_call(
        functools.partial(_vq_kernel, k_total=k),
        grid=grid,
        in_specs=[
            pl.BlockSpec((bn, d), lambda i: (i, 0)),
            pl.BlockSpec((k, d), lambda i: (0, 0)),
            pl.BlockSpec((bn, 1), lambda i: (i, 0)),
            pl.BlockSpec((1, k), lambda i: (0, 0)),
        ],
        out_specs=[
            pl.BlockSpec((bn, d), lambda i: (i, 0)),
            pl.BlockSpec((1, 1, 1), lambda i: (i, 0, 0)),
        ],
        out_shape=[
            jax.ShapeDtypeStruct((n, d), jnp.float32),
            jax.ShapeDtypeStruct((n // bn, 1, 1), jnp.float32),
        ],
        compiler_params=pltpu.CompilerParams(
            dimension_semantics=("parallel",)),
    )(weights_flat, W, x2, w2)

    mean_sq = jnp.sum(loss_parts) / (n * d)
    vq_loss = mean_sq + 0.1 * mean_sq
    return (out, vq_loss)
